# baseline (speedup 1.0000x reference)
import functools

import jax
import jax.numpy as jnp
from jax import lax
from jax.experimental import pallas as pl
from jax.experimental.pallas import tpu as pltpu
from jax.experimental.pallas import tpu_sc as plsc

_PW = 0.005
_RGB_W, _INTER_W, _DIST_W, _HASH_W = 1.0, 1.0, 0.01, 0.1
_HSIZE = 4096
_NW = 32
_SC_BLK = 16384


def _ray_body(s_ref, w_ref, ps0_ref, pw0_ref, ps1_ref, pw1_ref,
              pd_ref, gt_ref, out_ref):
    f32 = jnp.float32
    i = pl.program_id(0)

    @pl.when(i == 0)
    def _():
        out_ref[...] = jnp.zeros_like(out_ref)

    s = s_ref[...]
    w = w_ref[...]
    B = s.shape[1]
    zrow = jnp.zeros((1, B), f32)

    rwn = w / (s[1:] - s[:-1] + 1e-8)
    radio = (jnp.concatenate([rwn, zrow], 0)
             - jnp.concatenate([zrow, rwn], 0)) / (2.0 * _PW)
    a = s - _PW
    b = s + _PW
    C = (s[None, :, :] < (s[:, None, :] - 2.0 * _PW)).astype(f32)
    ii = lax.broadcasted_iota(jnp.int32, (33, B), 0).astype(f32)
    ra = ii + jnp.sum(C, axis=1)
    rb = ii + 33.0 - jnp.sum(C, axis=0)
    kk = lax.broadcasted_iota(jnp.int32, (33, 66, B), 1).astype(f32)
    oh_a = (ra[:, None, :] == kk).astype(f32)
    oh_b = (rb[:, None, :] == kk).astype(f32)
    bs = jnp.sum(a[:, None, :] * oh_a + b[:, None, :] * oh_b, axis=0)
    rs = jnp.sum(radio[:, None, :] * (oh_a[:, :65, :] - oh_b[:, :65, :]),
                 axis=0)
    dsd = bs[1:] - bs[:-1]
    lt65 = (lax.broadcasted_iota(jnp.int32, (65, 65), 0)
            >= lax.broadcasted_iota(jnp.int32, (65, 65), 1)).astype(f32)
    dot = functools.partial(jax.lax.dot_general,
                            dimension_numbers=(((1,), (0,)), ((), ())),
                            precision=jax.lax.Precision.HIGHEST,
                            preferred_element_type=f32)
    cr = dot(lt65, rs)
    recover = jnp.maximum(dot(lt65, dsd * cr), 0.0)
    bwf = jnp.concatenate([zrow, recover], 0)
    area = 0.5 * (bwf[1:] + bwf[:-1]) * dsd
    cdf = jnp.concatenate([zrow, dot(lt65, area)], 0)

    inter_p = jnp.zeros((B,), f32)
    for ps_ref, pw_ref in ((ps0_ref, pw0_ref), (ps1_ref, pw1_ref)):
        ps = ps_ref[...]
        pw = pw_ref[...]
        mask = ps[None, :, :] >= bs[:, None, :]
        mm = lax.broadcasted_iota(jnp.int32, (66, 65, B), 0).astype(f32)
        av = jnp.where(mask, cdf[:, None, :], cdf[0][None, None, :])
        x0 = jnp.max(av, axis=0)
        i0 = jnp.min(jnp.where(av == x0[None], mm, 66.0), axis=0)
        bv = jnp.where(mask, cdf[65][None, None, :], cdf[:, None, :])
        x1 = jnp.min(bv, axis=0)
        i1 = jnp.min(jnp.where(bv == x1[None], mm, 66.0), axis=0)
        fpdf0 = jnp.sum(jnp.where(i0[None] == mm, bwf[:, None, :], 0.0), axis=0)
        fpdf1 = jnp.sum(jnp.where(i1[None] == mm, bwf[:, None, :], 0.0), axis=0)
        axp = jnp.where(mask, bs[:, None, :], bs[0][None, None, :])
        xp0 = jnp.max(axp, axis=0)
        bxp = jnp.where(mask, bs[65][None, None, :], bs[:, None, :])
        xp1 = jnp.min(bxp, axis=0)
        z = (ps - xp0) / (xp1 - xp0)
        z = jnp.where(jnp.isnan(z), 0.0, z)
        off = jnp.clip(z, 0.0, 1.0)
        ci = x0 + (ps - xp0) * (fpdf0 + fpdf1 * off + fpdf0 * (1.0 - off)) / 2.0
        w_s = ci[1:] - ci[:-1]
        inter_p = inter_p + jnp.sum(
            jnp.maximum(w_s - pw, 0.0) ** 2 / (pw + 1e-5), axis=0)

    mids = (s[1:] + s[:-1]) / 2.0
    sd = jnp.abs(mids[:, None, :] - mids[None, :, :])
    t1 = jnp.sum(w[None, :, :] * sd, axis=1)
    p1 = jnp.sum(w * t1, axis=0)
    p2 = jnp.sum(w ** 2 * (s[1:] - s[:-1]), axis=0) / 3.0
    dist_p = jnp.abs(p1) + jnp.abs(p2)

    pd = pd_ref[...]
    gt = gt_ref[...]
    rgb_p = jnp.sum((pd - gt) ** 2, axis=0)

    pad = jnp.zeros((5, B), f32)
    out_ref[...] += jnp.concatenate(
        [rgb_p[None], inter_p[None], dist_p[None], pad], 0)


def _ray_losses(sT, wT, ps0T, pw0T, ps1T, pw1T, pdT, gtT):
    Rtot = sT.shape[1]
    BR = 128
    grid = Rtot // BR
    sp = lambda r: pl.BlockSpec((r, BR), lambda i: (0, i))
    return pl.pallas_call(
        _ray_body,
        grid=(grid,),
        in_specs=[sp(33), sp(32), sp(65), sp(64), sp(65), sp(64), sp(3), sp(3)],
        out_specs=pl.BlockSpec((8, 128), lambda i: (0, 0)),
        out_shape=jax.ShapeDtypeStruct((8, 128), jnp.float32),
    )(sT, wT, ps0T, pw0T, ps1T, pw1T, pdT, gtT)


def _sq_body(a_ref, b_ref, o_ref):
    a = a_ref[...]
    b = b_ref[...]
    o_ref[...] = a * a + b * b


def _sq_vals(c0, c1):
    R = c0.shape[0] // 128
    BR = R // 8
    sp = pl.BlockSpec((BR, 128), lambda i: (i, 0))
    return pl.pallas_call(
        _sq_body,
        grid=(8,),
        in_specs=[sp, sp],
        out_specs=sp,
        out_shape=jax.ShapeDtypeStruct((R, 128), jnp.float32),
    )(c0.reshape(R, 128), c1.reshape(R, 128)).reshape(-1)


def _hash_partials(v0, i0, v1, i1):
    N = i0.shape[0]
    M = N // _NW
    B = _SC_BLK
    nblk = M // B
    mesh = plsc.VectorSubcoreMesh(core_axis_name="c", subcore_axis_name="s")

    @functools.partial(
        pl.kernel, mesh=mesh,
        out_type=[jax.ShapeDtypeStruct((2, 2, _HSIZE), jnp.float32),
                  jax.ShapeDtypeStruct((2, 2, _HSIZE), jnp.float32)],
        scratch_types=[pltpu.VMEM((B,), jnp.int32),
                       pltpu.VMEM((B,), jnp.int32),
                       pltpu.VMEM((B,), jnp.float32),
                       pltpu.VMEM((B,), jnp.float32),
                       pltpu.VMEM((B,), jnp.float32),
                       pltpu.VMEM((_HSIZE,), jnp.float32),
                       pltpu.VMEM_SHARED((_HSIZE,), jnp.float32),
                       pltpu.VMEM_SHARED((_HSIZE,), jnp.float32),
                       pltpu.SemaphoreType.DMA,
                       pltpu.SemaphoreType.DMA,
                       pltpu.SemaphoreType.DMA,
                       pltpu.SemaphoreType.DMA,
                       pltpu.SemaphoreType.DMA,
                       pltpu.SemaphoreType.DMA,
                       pltpu.SemaphoreType.DMA,
                       pltpu.SemaphoreType.DMA],
    )
    def k(v0_h, i0_h, v1_h, i1_h, outS, outC,
          idx_a, idx_b, v_a, v_b, ones_v, zero_v,
          acc_sh, cnt_sh,
          si_a, si_b, s0_a, s0_b, sv_a, sv_b, so_a, so_b):
        cid = lax.axis_index("c")
        sid = lax.axis_index("s")
        wid = cid * 16 + sid
        base0 = wid * M
        ones16 = jnp.ones((16,), jnp.float32)
        zeros16 = jnp.zeros((16,), jnp.float32)

        def fill(v, _):
            ones_v[pl.ds(v * 16, 16)] = ones16
            return 0
        lax.fori_loop(0, B // 16, fill, 0)

        def fillz(v, _):
            zero_v[pl.ds(v * 16, 16)] = zeros16
            return 0
        lax.fori_loop(0, _HSIZE // 16, fillz, 0)

        idx_bufs = (idx_a, idx_b)
        v_bufs = (v_a, v_b)
        si = (si_a, si_b)
        s0 = (s0_a, s0_b)
        sv = (sv_a, sv_b)
        so = (so_a, so_b)

        for t, (v_h, i_h) in enumerate(((v0_h, i0_h), (v1_h, i1_h))):

            @pl.when(sid == 0)
            def _():
                pltpu.sync_copy(zero_v, acc_sh)
                pltpu.sync_copy(zero_v, cnt_sh)
            plsc.subcore_barrier()

            handles = {}

            def issue(blk, buf):
                st = base0 + blk * B
                handles[(buf, 0)] = pltpu.async_copy(
                    i_h.at[pl.ds(st, B)], idx_bufs[buf], si[buf])
                handles[(buf, 1)] = pltpu.async_copy(
                    v_h.at[pl.ds(st, B)], v_bufs[buf], s0[buf])

            issue(0, 0)
            for blk in range(nblk):
                buf = blk % 2
                handles[(buf, 0)].wait()
                handles[(buf, 1)].wait()
                if blk + 1 < nblk:
                    if blk >= 1:
                        handles[(1 - buf, 2)].wait()
                        handles[(1 - buf, 3)].wait()
                    issue(blk + 1, 1 - buf)
                handles[(buf, 2)] = pltpu.async_copy(
                    v_bufs[buf], acc_sh.at[idx_bufs[buf]], sv[buf], add=True)
                handles[(buf, 3)] = pltpu.async_copy(
                    ones_v, cnt_sh.at[idx_bufs[buf]], so[buf], add=True)

            handles[(nblk % 2, 2)].wait()
            handles[(nblk % 2, 3)].wait()
            handles[((nblk + 1) % 2, 2)].wait()
            handles[((nblk + 1) % 2, 3)].wait()
            plsc.subcore_barrier()

            @pl.when(sid == 0)
            def _():
                pltpu.sync_copy(acc_sh, outS.at[t, cid])
                pltpu.sync_copy(cnt_sh, outC.at[t, cid])
            plsc.subcore_barrier()

    return k(v0, i0, v1, i1)


def _hash_combine_body(s_ref, c_ref, o_ref):
    f32 = jnp.float32
    S = jnp.sum(s_ref[...], axis=1)
    C = jnp.sum(c_ref[...], axis=1)
    present = C > 0.0
    mean = jnp.where(present, S / jnp.maximum(C, 1.0), 0.0)
    ssum = jnp.sum(mean, axis=1)
    segid = lax.broadcasted_iota(jnp.int32, (2, _HSIZE), 1).astype(f32) + 1.0
    num_seg = jnp.max(jnp.where(present, segid, 0.0), axis=1)
    lh = jnp.sum(ssum / (num_seg * 2.0))
    o_ref[...] = jnp.full((1, 128), lh, f32)


def _hash_combine(Sp, Cp):
    return pl.pallas_call(
        _hash_combine_body,
        in_specs=[pl.BlockSpec((2, 2, _HSIZE), lambda: (0, 0, 0)),
                  pl.BlockSpec((2, 2, _HSIZE), lambda: (0, 0, 0))],
        out_specs=pl.BlockSpec((1, 128), lambda: (0, 0)),
        out_shape=jax.ShapeDtypeStruct((1, 128), jnp.float32),
    )(Sp, Cp)


def kernel(pd_rgbs, gt_rgbs, render_sdist, render_weights,
           prop_sdist_0, prop_weights_0, prop_sdist_1, prop_weights_1,
           enc_embds_0, enc_idx_0, enc_embds_1, enc_idx_1):
    Rtot = pd_rgbs.shape[0]
    ray_out = _ray_losses(render_sdist.T, render_weights.T,
                          prop_sdist_0.T, prop_weights_0.T,
                          prop_sdist_1.T, prop_weights_1.T,
                          pd_rgbs.T, gt_rgbs.T)
    v0 = _sq_vals(enc_embds_0[:, 0], enc_embds_0[:, 1])
    v1 = _sq_vals(enc_embds_1[:, 0], enc_embds_1[:, 1])
    Sp, Cp = _hash_partials(v0, enc_idx_0.astype(jnp.int32),
                            v1, enc_idx_1.astype(jnp.int32))
    lh = _hash_combine(Sp, Cp)[0, 0]
    loss_rgb = jnp.sum(ray_out[0]) / (Rtot * 3)
    loss_inter = jnp.sum(ray_out[1]) / (Rtot * 64)
    loss_dist = jnp.sum(ray_out[2]) / Rtot
    return (_RGB_W * loss_rgb + _INTER_W * loss_inter
            + _DIST_W * loss_dist + _HASH_W * lh)


# speedup vs baseline: 1.1405x; 1.1405x over previous
import functools

import jax
import jax.numpy as jnp
from jax import lax
from jax.experimental import pallas as pl
from jax.experimental.pallas import tpu as pltpu
from jax.experimental.pallas import tpu_sc as plsc

_PW = 0.005
_RGB_W, _INTER_W, _DIST_W, _HASH_W = 1.0, 1.0, 0.01, 0.1
_HSIZE = 4096
_NW = 32
_SC_BLK = 8192


def _ray_body(s_ref, w_ref, ps0_ref, pw0_ref, ps1_ref, pw1_ref,
              pd_ref, gt_ref, out_ref):
    f32 = jnp.float32
    i = pl.program_id(0)

    @pl.when(i == 0)
    def _():
        out_ref[...] = jnp.zeros_like(out_ref)

    s = s_ref[...]
    w = w_ref[...]
    B = s.shape[1]
    zrow = jnp.zeros((1, B), f32)

    rwn = w / (s[1:] - s[:-1] + 1e-8)
    radio = (jnp.concatenate([rwn, zrow], 0)
             - jnp.concatenate([zrow, rwn], 0)) / (2.0 * _PW)
    a = s - _PW
    b = s + _PW
    C = (s[None, :, :] < (s[:, None, :] - 2.0 * _PW)).astype(f32)
    ii = lax.broadcasted_iota(jnp.int32, (33, B), 0).astype(f32)
    ra = ii + jnp.sum(C, axis=1)
    rb = ii + 33.0 - jnp.sum(C, axis=0)
    kk = lax.broadcasted_iota(jnp.int32, (33, 66, B), 1).astype(f32)
    oh_a = (ra[:, None, :] == kk).astype(f32)
    oh_b = (rb[:, None, :] == kk).astype(f32)
    bs = jnp.sum(a[:, None, :] * oh_a + b[:, None, :] * oh_b, axis=0)
    rs = jnp.sum(radio[:, None, :] * (oh_a[:, :65, :] - oh_b[:, :65, :]),
                 axis=0)
    dsd = bs[1:] - bs[:-1]
    lt65 = (lax.broadcasted_iota(jnp.int32, (65, 65), 0)
            >= lax.broadcasted_iota(jnp.int32, (65, 65), 1)).astype(f32)
    dot = functools.partial(jax.lax.dot_general,
                            dimension_numbers=(((1,), (0,)), ((), ())),
                            precision=jax.lax.Precision.HIGHEST,
                            preferred_element_type=f32)
    cr = dot(lt65, rs)
    recover = jnp.maximum(dot(lt65, dsd * cr), 0.0)
    bwf = jnp.concatenate([zrow, recover], 0)
    area = 0.5 * (bwf[1:] + bwf[:-1]) * dsd
    cdf = jnp.concatenate([zrow, dot(lt65, area)], 0)

    inter_p = jnp.zeros((B,), f32)
    for ps_ref, pw_ref in ((ps0_ref, pw0_ref), (ps1_ref, pw1_ref)):
        ps = ps_ref[...]
        pw = pw_ref[...]
        mask = ps[None, :, :] >= bs[:, None, :]
        mm = lax.broadcasted_iota(jnp.int32, (66, 65, B), 0).astype(f32)
        av = jnp.where(mask, cdf[:, None, :], cdf[0][None, None, :])
        x0 = jnp.max(av, axis=0)
        i0 = jnp.min(jnp.where(av == x0[None], mm, 66.0), axis=0)
        bv = jnp.where(mask, cdf[65][None, None, :], cdf[:, None, :])
        x1 = jnp.min(bv, axis=0)
        i1 = jnp.min(jnp.where(bv == x1[None], mm, 66.0), axis=0)
        fpdf0 = jnp.sum(jnp.where(i0[None] == mm, bwf[:, None, :], 0.0), axis=0)
        fpdf1 = jnp.sum(jnp.where(i1[None] == mm, bwf[:, None, :], 0.0), axis=0)
        axp = jnp.where(mask, bs[:, None, :], bs[0][None, None, :])
        xp0 = jnp.max(axp, axis=0)
        bxp = jnp.where(mask, bs[65][None, None, :], bs[:, None, :])
        xp1 = jnp.min(bxp, axis=0)
        z = (ps - xp0) / (xp1 - xp0)
        z = jnp.where(jnp.isnan(z), 0.0, z)
        off = jnp.clip(z, 0.0, 1.0)
        ci = x0 + (ps - xp0) * (fpdf0 + fpdf1 * off + fpdf0 * (1.0 - off)) / 2.0
        w_s = ci[1:] - ci[:-1]
        inter_p = inter_p + jnp.sum(
            jnp.maximum(w_s - pw, 0.0) ** 2 / (pw + 1e-5), axis=0)

    mids = (s[1:] + s[:-1]) / 2.0
    sd = jnp.abs(mids[:, None, :] - mids[None, :, :])
    t1 = jnp.sum(w[None, :, :] * sd, axis=1)
    p1 = jnp.sum(w * t1, axis=0)
    p2 = jnp.sum(w ** 2 * (s[1:] - s[:-1]), axis=0) / 3.0
    dist_p = jnp.abs(p1) + jnp.abs(p2)

    pd = pd_ref[...]
    gt = gt_ref[...]
    rgb_p = jnp.sum((pd - gt) ** 2, axis=0)

    pad = jnp.zeros((5, B), f32)
    out_ref[...] += jnp.concatenate(
        [rgb_p[None], inter_p[None], dist_p[None], pad], 0)


def _ray_losses(sT, wT, ps0T, pw0T, ps1T, pw1T, pdT, gtT):
    Rtot = sT.shape[1]
    BR = 128
    grid = Rtot // BR
    sp = lambda r: pl.BlockSpec((r, BR), lambda i: (0, i))
    return pl.pallas_call(
        _ray_body,
        grid=(grid,),
        in_specs=[sp(33), sp(32), sp(65), sp(64), sp(65), sp(64), sp(3), sp(3)],
        out_specs=pl.BlockSpec((8, 128), lambda i: (0, 0)),
        out_shape=jax.ShapeDtypeStruct((8, 128), jnp.float32),
    )(sT, wT, ps0T, pw0T, ps1T, pw1T, pdT, gtT)


def _hash_partials(e0c0, e0c1, i0, e1c0, e1c1, i1):
    N = i0.shape[0]
    M = N // _NW
    B = _SC_BLK
    nblk = M // B
    mesh = plsc.VectorSubcoreMesh(core_axis_name="c", subcore_axis_name="s")

    @functools.partial(
        pl.kernel, mesh=mesh,
        out_type=[jax.ShapeDtypeStruct((2, 2, _HSIZE), jnp.float32),
                  jax.ShapeDtypeStruct((2, 2, _HSIZE), jnp.float32)],
        scratch_types=[pltpu.VMEM((B,), jnp.int32),
                       pltpu.VMEM((B,), jnp.int32),
                       pltpu.VMEM((B,), jnp.float32),
                       pltpu.VMEM((B,), jnp.float32),
                       pltpu.VMEM((B,), jnp.float32),
                       pltpu.VMEM((B,), jnp.float32),
                       pltpu.VMEM((B,), jnp.float32),
                       pltpu.VMEM((_HSIZE,), jnp.float32),
                       pltpu.VMEM_SHARED((_HSIZE,), jnp.float32),
                       pltpu.VMEM_SHARED((_HSIZE,), jnp.float32),
                       pltpu.SemaphoreType.DMA,
                       pltpu.SemaphoreType.DMA,
                       pltpu.SemaphoreType.DMA,
                       pltpu.SemaphoreType.DMA,
                       pltpu.SemaphoreType.DMA,
                       pltpu.SemaphoreType.DMA],
    )
    def k(e0c0_h, e0c1_h, i0_h, e1c0_h, e1c1_h, i1_h, outS, outC,
          idx_a, idx_b, c0_a, c0_b, c1_a, c1_b, ones_v, zero_v,
          acc_sh, cnt_sh,
          si_a, si_b, s0_a, s0_b, s1_a, s1_b):
        cid = lax.axis_index("c")
        sid = lax.axis_index("s")
        wid = cid * 16 + sid
        base0 = wid * M
        ones16 = jnp.ones((16,), jnp.float32)
        zeros16 = jnp.zeros((16,), jnp.float32)

        def fill(v, _):
            ones_v[pl.ds(v * 16, 16)] = ones16
            return 0
        lax.fori_loop(0, B // 16, fill, 0)

        def fillz(v, _):
            zero_v[pl.ds(v * 16, 16)] = zeros16
            return 0
        lax.fori_loop(0, _HSIZE // 16, fillz, 0)

        idx_bufs = (idx_a, idx_b)
        c0_bufs = (c0_a, c0_b)
        c1_bufs = (c1_a, c1_b)
        si = (si_a, si_b)
        s0 = (s0_a, s0_b)
        s1 = (s1_a, s1_b)

        for t, (ec0_h, ec1_h, i_h) in enumerate(((e0c0_h, e0c1_h, i0_h),
                                                 (e1c0_h, e1c1_h, i1_h))):

            @pl.when(sid == 0)
            def _():
                pltpu.sync_copy(zero_v, acc_sh)
                pltpu.sync_copy(zero_v, cnt_sh)
            plsc.subcore_barrier()

            handles = {}

            def issue(blk, buf):
                st = base0 + blk * B
                handles[(buf, 0)] = pltpu.async_copy(
                    i_h.at[pl.ds(st, B)], idx_bufs[buf], si[buf])
                handles[(buf, 1)] = pltpu.async_copy(
                    ec0_h.at[pl.ds(st, B)], c0_bufs[buf], s0[buf])
                handles[(buf, 2)] = pltpu.async_copy(
                    ec1_h.at[pl.ds(st, B)], c1_bufs[buf], s1[buf])

            issue(0, 0)
            for blk in range(nblk):
                buf = blk % 2
                handles[(buf, 0)].wait()
                handles[(buf, 1)].wait()
                handles[(buf, 2)].wait()
                if blk + 1 < nblk:
                    issue(blk + 1, 1 - buf)
                c0r = c0_bufs[buf]
                c1r = c1_bufs[buf]

                def body(v, _):
                    sl = pl.ds(v * 16, 16)
                    a = c0r[sl]
                    b = c1r[sl]
                    c0r[sl] = a * a + b * b
                    return 0
                lax.fori_loop(0, B // 16, body, 0)

                pltpu.sync_copy(c0r, acc_sh.at[idx_bufs[buf]], add=True)
                pltpu.sync_copy(ones_v, cnt_sh.at[idx_bufs[buf]], add=True)

            plsc.subcore_barrier()

            @pl.when(sid == 0)
            def _():
                pltpu.sync_copy(acc_sh, outS.at[t, cid])
                pltpu.sync_copy(cnt_sh, outC.at[t, cid])
            plsc.subcore_barrier()

    return k(e0c0, e0c1, i0, e1c0, e1c1, i1)


def _hash_combine_body(s_ref, c_ref, o_ref):
    f32 = jnp.float32
    S = jnp.sum(s_ref[...], axis=1)
    C = jnp.sum(c_ref[...], axis=1)
    present = C > 0.0
    mean = jnp.where(present, S / jnp.maximum(C, 1.0), 0.0)
    ssum = jnp.sum(mean, axis=1)
    segid = lax.broadcasted_iota(jnp.int32, (2, _HSIZE), 1).astype(f32) + 1.0
    num_seg = jnp.max(jnp.where(present, segid, 0.0), axis=1)
    lh = jnp.sum(ssum / (num_seg * 2.0))
    o_ref[...] = jnp.full((1, 128), lh, f32)


def _hash_combine(Sp, Cp):
    return pl.pallas_call(
        _hash_combine_body,
        in_specs=[pl.BlockSpec((2, 2, _HSIZE), lambda: (0, 0, 0)),
                  pl.BlockSpec((2, 2, _HSIZE), lambda: (0, 0, 0))],
        out_specs=pl.BlockSpec((1, 128), lambda: (0, 0)),
        out_shape=jax.ShapeDtypeStruct((1, 128), jnp.float32),
    )(Sp, Cp)


def kernel(pd_rgbs, gt_rgbs, render_sdist, render_weights,
           prop_sdist_0, prop_weights_0, prop_sdist_1, prop_weights_1,
           enc_embds_0, enc_idx_0, enc_embds_1, enc_idx_1):
    Rtot = pd_rgbs.shape[0]
    ray_out = _ray_losses(render_sdist.T, render_weights.T,
                          prop_sdist_0.T, prop_weights_0.T,
                          prop_sdist_1.T, prop_weights_1.T,
                          pd_rgbs.T, gt_rgbs.T)
    Sp, Cp = _hash_partials(enc_embds_0[:, 0], enc_embds_0[:, 1],
                            enc_idx_0.astype(jnp.int32),
                            enc_embds_1[:, 0], enc_embds_1[:, 1],
                            enc_idx_1.astype(jnp.int32))
    lh = _hash_combine(Sp, Cp)[0, 0]
    loss_rgb = jnp.sum(ray_out[0]) / (Rtot * 3)
    loss_inter = jnp.sum(ray_out[1]) / (Rtot * 64)
    loss_dist = jnp.sum(ray_out[2]) / Rtot
    return (_RGB_W * loss_rgb + _INTER_W * loss_inter
            + _DIST_W * loss_dist + _HASH_W * lh)
